# baseline (speedup 1.0000x reference)
import jax
import jax.numpy as jnp
from jax.experimental import pallas as pl
from jax.experimental.pallas import tpu as pltpu

_EPOCHS = 200
_DECAY_W = 1.0
_TH = 0.8
_INCREMENT = 0.5 / _EPOCHS
_CO_LAMBDA = 0.1

_R = 1024


def _phase1_kernel(y1_ref, y2_ref, t_ref, out_ref):
    y1 = y1_ref[...]
    y2 = y2_ref[...]
    t = t_ref[0]
    C = y1.shape[1]
    ones_f = jnp.ones((C, 1), jnp.float32)

    m1 = jnp.max(y1, axis=1, keepdims=True)
    m2 = jnp.max(y2, axis=1, keepdims=True)
    e1 = jnp.exp(y1 - m1)
    e2 = jnp.exp(y2 - m2)
    s1 = jnp.dot(e1, ones_f, preferred_element_type=jnp.float32)
    s2 = jnp.dot(e2, ones_f, preferred_element_type=jnp.float32)
    l1 = jnp.log(s1)
    l2 = jnp.log(s2)

    iota = jax.lax.broadcasted_iota(jnp.int32, y1.shape, 1)
    onehot_t = iota == t
    y1t = jnp.dot(
        jnp.where(onehot_t, y1, 0.0), ones_f,
        preferred_element_type=jnp.float32,
    )
    y2t = jnp.dot(
        jnp.where(onehot_t, y2, 0.0), ones_f,
        preferred_element_type=jnp.float32,
    )
    g1 = y1t - m1
    g2 = y2t - m2

    loss = jnp.maximum((l1 - g1) + (l2 - g2), 0.0)
    dc = l1 + l2

    agree = jnp.dot(
        jnp.where(y1 == m1, e2, 0.0), ones_f,
        preferred_element_type=jnp.float32,
    )
    pmax_prod = (1.0 / s1) * (1.0 / s2)
    cond = jnp.logical_and(
        jnp.logical_and(g1 != 0.0, agree == 1.0), pmax_prod > _TH * _TH
    )

    d = y1 - y2
    S1 = jnp.dot(e1 * d, ones_f, preferred_element_type=jnp.float32)
    S2 = jnp.dot(e2 * d, ones_f, preferred_element_type=jnp.float32)
    s_row = S1 / s1 - S2 / s2

    out_ref[0, 0:1, :] = loss.T
    out_ref[0, 1:2, :] = dc.T
    out_ref[0, 2:3, :] = cond.astype(jnp.float32).T
    out_ref[0, 3:4, :] = s_row.T


def _phase2_kernel(packed_ref, kfloor_ref, out_ref):
    loss = packed_ref[:, 0, :]
    RR, CC = loss.shape
    n = RR * CC
    n_f = jnp.float32(n)

    sum_loss = jnp.sum(loss)
    mean_v = sum_loss / n_f
    n_small = jnp.sum((loss < mean_v).astype(jnp.int32))
    k = jnp.maximum(kfloor_ref[0, 0], n_small)

    bits = jax.lax.bitcast_convert_type(loss, jnp.int32)

    def body(_, carry):
        lo, hi = carry
        mid = lo + (hi - lo) // 2
        c = jnp.sum((bits <= mid).astype(jnp.int32))
        pred = c >= k
        return jnp.where(pred, lo, mid + 1), jnp.where(pred, mid, hi)

    _, vbits = jax.lax.fori_loop(
        0, 31, body, (jnp.int32(0), jnp.int32(0x7F800000))
    )

    count_less = jnp.sum((bits < vbits).astype(jnp.int32))
    eq = bits == vbits
    need = (k - count_less).astype(jnp.float32)

    eqf = eq.astype(jnp.float32)
    r0 = jax.lax.broadcasted_iota(jnp.int32, (RR, CC), 0)
    c1 = jax.lax.broadcasted_iota(jnp.int32, (RR, CC), 1)
    lc0 = jax.lax.broadcasted_iota(jnp.int32, (CC, CC), 0)
    lc1 = jax.lax.broadcasted_iota(jnp.int32, (CC, CC), 1)
    lower_incl = (lc0 <= lc1).astype(jnp.float32)
    lr0 = jax.lax.broadcasted_iota(jnp.int32, (RR, RR), 0)
    lr1 = jax.lax.broadcasted_iota(jnp.int32, (RR, RR), 1)
    strict_lower = (lr1 < lr0).astype(jnp.float32)
    incl = jnp.dot(eqf, lower_incl, preferred_element_type=jnp.float32)
    rowtot = incl[:, CC - 1 : CC]
    offs = jnp.dot(strict_lower, rowtot, preferred_element_type=jnp.float32)
    eq_before = incl - eqf + offs

    in_update = (bits < vbits) | (eq & (eq_before < need))

    flat_idx = r0 * CC + c1
    upd1 = jnp.logical_and(jnp.logical_not(in_update), flat_idx >= 1)
    condb = packed_ref[:, 2, :] > 0.5

    loss_clean = jnp.sum(jnp.where(in_update, loss, 0.0))
    loss_dc = jnp.sum(
        jnp.where(jnp.logical_and(upd1, condb), packed_ref[:, 1, :], 0.0)
    )
    loss1 = jnp.sum(
        jnp.where(jnp.logical_and(upd1, jnp.logical_not(condb)), loss, 0.0)
    )
    inter = jnp.sum(packed_ref[:, 3, :])

    out_ref[0, 0] = (
        loss_clean + loss_dc + _DECAY_W * loss1
    ) / n_f + _CO_LAMBDA * (inter / n_f)


def kernel(y_1, y_2, t, epoch):
    N, C = y_1.shape
    G = N // _R

    t3 = t.reshape(G, _R, 1)
    packed = pl.pallas_call(
        _phase1_kernel,
        grid=(G,),
        in_specs=[
            pl.BlockSpec((_R, C), lambda i: (i, 0)),
            pl.BlockSpec((_R, C), lambda i: (i, 0)),
            pl.BlockSpec((1, _R, 1), lambda i: (i, 0, 0)),
        ],
        out_specs=pl.BlockSpec((1, 4, _R), lambda i: (i, 0, 0)),
        out_shape=jax.ShapeDtypeStruct((G, 4, _R), jnp.float32),
    )(y_1, y_2, t3)

    remember_rate = 1.0 - _INCREMENT * epoch
    kfloor = jnp.floor(remember_rate * N).astype(jnp.int32).reshape(1, 1)

    out = pl.pallas_call(
        _phase2_kernel,
        in_specs=[
            pl.BlockSpec((G, 4, _R), lambda: (0, 0, 0)),
            pl.BlockSpec(memory_space=pltpu.SMEM),
        ],
        out_specs=pl.BlockSpec(memory_space=pltpu.SMEM),
        out_shape=jax.ShapeDtypeStruct((1, 1), jnp.float32),
    )(packed, kfloor)

    return out.reshape(())


# speedup vs baseline: 1.0295x; 1.0295x over previous
import jax
import jax.numpy as jnp
from jax.experimental import pallas as pl
from jax.experimental.pallas import tpu as pltpu

_EPOCHS = 200
_DECAY_W = 1.0
_TH = 0.8
_INCREMENT = 0.5 / _EPOCHS
_CO_LAMBDA = 0.1

_R = 1024


def _phase1_kernel(y1_ref, y2_ref, t_ref, out_ref):
    y1 = y1_ref[...]
    y2 = y2_ref[...]
    t = t_ref[0]
    C = y1.shape[1]
    ones_f = jnp.ones((C, 1), jnp.float32)

    m1 = jnp.max(y1, axis=1, keepdims=True)
    m2 = jnp.max(y2, axis=1, keepdims=True)
    e1 = jnp.exp(y1 - m1)
    e2 = jnp.exp(y2 - m2)
    s1 = jnp.dot(e1, ones_f, preferred_element_type=jnp.float32)
    s2 = jnp.dot(e2, ones_f, preferred_element_type=jnp.float32)
    l1 = jnp.log(s1)
    l2 = jnp.log(s2)

    iota = jax.lax.broadcasted_iota(jnp.int32, y1.shape, 1)
    onehot_t = iota == t
    y1t = jnp.dot(
        jnp.where(onehot_t, y1, 0.0), ones_f,
        preferred_element_type=jnp.float32,
    )
    y2t = jnp.dot(
        jnp.where(onehot_t, y2, 0.0), ones_f,
        preferred_element_type=jnp.float32,
    )
    g1 = y1t - m1
    g2 = y2t - m2

    loss = jnp.maximum((l1 - g1) + (l2 - g2), 0.0)
    dc = l1 + l2

    agree = jnp.dot(
        jnp.where(y1 == m1, e2, 0.0), ones_f,
        preferred_element_type=jnp.float32,
    )
    pmax_prod = (1.0 / s1) * (1.0 / s2)
    cond = jnp.logical_and(
        jnp.logical_and(g1 != 0.0, agree == 1.0), pmax_prod > _TH * _TH
    )

    d = y1 - y2
    S1 = jnp.dot(e1 * d, ones_f, preferred_element_type=jnp.float32)
    S2 = jnp.dot(e2 * d, ones_f, preferred_element_type=jnp.float32)
    s_row = S1 / s1 - S2 / s2

    out_ref[0, 0:1, :] = loss.T
    out_ref[0, 1:2, :] = dc.T
    out_ref[0, 2:3, :] = cond.astype(jnp.float32).T
    out_ref[0, 3:4, :] = s_row.T


def _phase2_kernel(packed_ref, kfloor_ref, out_ref):
    g_, _, r_ = packed_ref.shape
    RR = 128
    CC = (g_ * r_) // RR
    loss = packed_ref[:, 0, :].reshape(RR, CC)
    n = RR * CC
    n_f = jnp.float32(n)

    sum_loss = jnp.sum(loss)
    mean_v = sum_loss / n_f
    n_small = jnp.sum((loss < mean_v).astype(jnp.int32))
    k = jnp.maximum(kfloor_ref[0, 0], n_small)

    bits = jax.lax.bitcast_convert_type(loss, jnp.int32)

    def body(_, carry):
        lo, hi = carry
        mid = lo + (hi - lo) // 2
        c = jnp.sum((bits <= mid).astype(jnp.int32))
        pred = c >= k
        return jnp.where(pred, lo, mid + 1), jnp.where(pred, mid, hi)

    _, vbits = jax.lax.fori_loop(
        0, 31, body, (jnp.int32(0), jnp.int32(0x7F800000))
    )

    count_less = jnp.sum((bits < vbits).astype(jnp.int32))
    eq = bits == vbits
    need = (k - count_less).astype(jnp.float32)

    eqf = eq.astype(jnp.float32)
    r0 = jax.lax.broadcasted_iota(jnp.int32, (RR, CC), 0)
    c1 = jax.lax.broadcasted_iota(jnp.int32, (RR, CC), 1)
    lower_incl = (r0 <= c1).astype(jnp.float32)
    strict_lower = (c1 < r0).astype(jnp.float32)
    incl = jnp.dot(eqf, lower_incl, preferred_element_type=jnp.float32)
    rowtot = incl[:, CC - 1 : CC]
    offs = jnp.dot(strict_lower, rowtot, preferred_element_type=jnp.float32)
    eq_before = incl - eqf + offs

    in_update = (bits < vbits) | (eq & (eq_before < need))

    flat_idx = r0 * CC + c1
    upd1 = jnp.logical_and(jnp.logical_not(in_update), flat_idx >= 1)
    condb = packed_ref[:, 2, :].reshape(RR, CC) > 0.5

    loss_clean = jnp.sum(jnp.where(in_update, loss, 0.0))
    loss_dc = jnp.sum(
        jnp.where(
            jnp.logical_and(upd1, condb),
            packed_ref[:, 1, :].reshape(RR, CC), 0.0,
        )
    )
    loss1 = jnp.sum(
        jnp.where(jnp.logical_and(upd1, jnp.logical_not(condb)), loss, 0.0)
    )
    inter = jnp.sum(packed_ref[:, 3, :])

    out_ref[0, 0] = (
        loss_clean + loss_dc + _DECAY_W * loss1
    ) / n_f + _CO_LAMBDA * (inter / n_f)


def kernel(y_1, y_2, t, epoch):
    N, C = y_1.shape
    G = N // _R

    t3 = t.reshape(G, _R, 1)
    packed = pl.pallas_call(
        _phase1_kernel,
        grid=(G,),
        in_specs=[
            pl.BlockSpec((_R, C), lambda i: (i, 0)),
            pl.BlockSpec((_R, C), lambda i: (i, 0)),
            pl.BlockSpec((1, _R, 1), lambda i: (i, 0, 0)),
        ],
        out_specs=pl.BlockSpec((1, 4, _R), lambda i: (i, 0, 0)),
        out_shape=jax.ShapeDtypeStruct((G, 4, _R), jnp.float32),
    )(y_1, y_2, t3)

    remember_rate = 1.0 - _INCREMENT * epoch
    kfloor = jnp.floor(remember_rate * N).astype(jnp.int32).reshape(1, 1)

    out = pl.pallas_call(
        _phase2_kernel,
        in_specs=[
            pl.BlockSpec((G, 4, _R), lambda: (0, 0, 0)),
            pl.BlockSpec(memory_space=pltpu.SMEM),
        ],
        out_specs=pl.BlockSpec(memory_space=pltpu.SMEM),
        out_shape=jax.ShapeDtypeStruct((1, 1), jnp.float32),
    )(packed, kfloor)

    return out.reshape(())


# speedup vs baseline: 1.0778x; 1.0469x over previous
import jax
import jax.numpy as jnp
from jax.experimental import pallas as pl
from jax.experimental.pallas import tpu as pltpu

_EPOCHS = 200
_DECAY_W = 1.0
_TH = 0.8
_INCREMENT = 0.5 / _EPOCHS
_CO_LAMBDA = 0.1

_R = 1024


def _phase1_kernel(y1_ref, y2_ref, t_ref, out_ref):
    y1 = y1_ref[...]
    y2 = y2_ref[...]
    t = t_ref[0]
    C = y1.shape[1]
    ones_f = jnp.ones((C, 1), jnp.float32)

    m1 = jnp.max(y1, axis=1, keepdims=True)
    m2 = jnp.max(y2, axis=1, keepdims=True)
    e1 = jnp.exp(y1 - m1)
    e2 = jnp.exp(y2 - m2)
    s1 = jnp.sum(e1, axis=1, keepdims=True)
    s2 = jnp.sum(e2, axis=1, keepdims=True)
    l1 = jnp.log(s1)
    l2 = jnp.log(s2)

    iota = jax.lax.broadcasted_iota(jnp.int32, y1.shape, 1)
    onehot_t = iota == t
    y1t = jnp.sum(jnp.where(onehot_t, y1, 0.0), axis=1, keepdims=True)
    y2t = jnp.sum(jnp.where(onehot_t, y2, 0.0), axis=1, keepdims=True)
    g1 = y1t - m1
    g2 = y2t - m2

    loss = jnp.maximum((l1 - g1) + (l2 - g2), 0.0)
    dc = l1 + l2

    agree = jnp.sum(jnp.where(y1 == m1, e2, 0.0), axis=1, keepdims=True)
    pmax_prod = (1.0 / s1) * (1.0 / s2)
    cond = jnp.logical_and(
        jnp.logical_and(g1 != 0.0, agree == 1.0), pmax_prod > _TH * _TH
    )

    d = y1 - y2
    S1 = jnp.sum(e1 * d, axis=1, keepdims=True)
    S2 = jnp.sum(e2 * d, axis=1, keepdims=True)
    s_row = S1 / s1 - S2 / s2

    out_ref[0, 0:1, :] = loss.T
    out_ref[0, 1:2, :] = dc.T
    out_ref[0, 2:3, :] = cond.astype(jnp.float32).T
    out_ref[0, 3:4, :] = s_row.T


def _phase2_kernel(packed_ref, kfloor_ref, out_ref):
    g_, _, r_ = packed_ref.shape
    RR = 128
    CC = (g_ * r_) // RR
    loss = packed_ref[:, 0, :].reshape(RR, CC)
    n = RR * CC
    n_f = jnp.float32(n)

    sum_loss = jnp.sum(loss)
    mean_v = sum_loss / n_f
    n_small = jnp.sum((loss < mean_v).astype(jnp.int32))
    k = jnp.maximum(kfloor_ref[0, 0], n_small)

    bits = jax.lax.bitcast_convert_type(loss, jnp.int32)

    def body(_, carry):
        lo, hi = carry
        mid = lo + (hi - lo) // 2
        c = jnp.sum((bits <= mid).astype(jnp.int32))
        pred = c >= k
        return jnp.where(pred, lo, mid + 1), jnp.where(pred, mid, hi)

    _, vbits = jax.lax.fori_loop(
        0, 31, body, (jnp.int32(0), jnp.int32(0x7F800000))
    )

    count_less = jnp.sum((bits < vbits).astype(jnp.int32))
    eq = bits == vbits
    need = (k - count_less).astype(jnp.float32)

    eqf = eq.astype(jnp.float32)
    r0 = jax.lax.broadcasted_iota(jnp.int32, (RR, CC), 0)
    c1 = jax.lax.broadcasted_iota(jnp.int32, (RR, CC), 1)
    lower_incl = (r0 <= c1).astype(jnp.float32)
    strict_lower = (c1 < r0).astype(jnp.float32)
    incl = jnp.dot(eqf, lower_incl, preferred_element_type=jnp.float32)
    rowtot = incl[:, CC - 1 : CC]
    offs = jnp.dot(strict_lower, rowtot, preferred_element_type=jnp.float32)
    eq_before = incl - eqf + offs

    in_update = (bits < vbits) | (eq & (eq_before < need))

    flat_idx = r0 * CC + c1
    upd1 = jnp.logical_and(jnp.logical_not(in_update), flat_idx >= 1)
    condb = packed_ref[:, 2, :].reshape(RR, CC) > 0.5

    loss_clean = jnp.sum(jnp.where(in_update, loss, 0.0))
    loss_dc = jnp.sum(
        jnp.where(
            jnp.logical_and(upd1, condb),
            packed_ref[:, 1, :].reshape(RR, CC), 0.0,
        )
    )
    loss1 = jnp.sum(
        jnp.where(jnp.logical_and(upd1, jnp.logical_not(condb)), loss, 0.0)
    )
    inter = jnp.sum(packed_ref[:, 3, :])

    out_ref[0, 0] = (
        loss_clean + loss_dc + _DECAY_W * loss1
    ) / n_f + _CO_LAMBDA * (inter / n_f)


def kernel(y_1, y_2, t, epoch):
    N, C = y_1.shape
    G = N // _R

    t3 = t.reshape(G, _R, 1)
    packed = pl.pallas_call(
        _phase1_kernel,
        grid=(G,),
        in_specs=[
            pl.BlockSpec((_R, C), lambda i: (i, 0)),
            pl.BlockSpec((_R, C), lambda i: (i, 0)),
            pl.BlockSpec((1, _R, 1), lambda i: (i, 0, 0)),
        ],
        out_specs=pl.BlockSpec((1, 4, _R), lambda i: (i, 0, 0)),
        out_shape=jax.ShapeDtypeStruct((G, 4, _R), jnp.float32),
    )(y_1, y_2, t3)

    remember_rate = 1.0 - _INCREMENT * epoch
    kfloor = jnp.floor(remember_rate * N).astype(jnp.int32).reshape(1, 1)

    out = pl.pallas_call(
        _phase2_kernel,
        in_specs=[
            pl.BlockSpec((G, 4, _R), lambda: (0, 0, 0)),
            pl.BlockSpec(memory_space=pltpu.SMEM),
        ],
        out_specs=pl.BlockSpec(memory_space=pltpu.SMEM),
        out_shape=jax.ShapeDtypeStruct((1, 1), jnp.float32),
    )(packed, kfloor)

    return out.reshape(())


# speedup vs baseline: 1.0919x; 1.0131x over previous
import jax
import jax.numpy as jnp
from jax.experimental import pallas as pl
from jax.experimental.pallas import tpu as pltpu

_EPOCHS = 200
_DECAY_W = 1.0
_TH = 0.8
_INCREMENT = 0.5 / _EPOCHS
_CO_LAMBDA = 0.1

_R = 1024


def _fused_kernel(y1_ref, y2_ref, t_ref, kfloor_ref, out_ref, acc_ref):
    i = pl.program_id(0)
    G = pl.num_programs(0)
    y1 = y1_ref[...]
    y2 = y2_ref[...]
    t = t_ref[0]
    C = y1.shape[1]

    m1 = jnp.max(y1, axis=1, keepdims=True)
    m2 = jnp.max(y2, axis=1, keepdims=True)
    e1 = jnp.exp(y1 - m1)
    e2 = jnp.exp(y2 - m2)
    s1 = jnp.sum(e1, axis=1, keepdims=True)
    s2 = jnp.sum(e2, axis=1, keepdims=True)
    l1 = jnp.log(s1)
    l2 = jnp.log(s2)

    iota = jax.lax.broadcasted_iota(jnp.int32, y1.shape, 1)
    onehot_t = iota == t
    y1t = jnp.sum(jnp.where(onehot_t, y1, 0.0), axis=1, keepdims=True)
    y2t = jnp.sum(jnp.where(onehot_t, y2, 0.0), axis=1, keepdims=True)
    g1 = y1t - m1
    g2 = y2t - m2

    loss_r = jnp.maximum((l1 - g1) + (l2 - g2), 0.0)
    dc_r = l1 + l2

    agree = jnp.sum(jnp.where(y1 == m1, e2, 0.0), axis=1, keepdims=True)
    pmax_prod = (1.0 / s1) * (1.0 / s2)
    cond_r = jnp.logical_and(
        jnp.logical_and(g1 != 0.0, agree == 1.0), pmax_prod > _TH * _TH
    )

    d = y1 - y2
    S1 = jnp.sum(e1 * d, axis=1, keepdims=True)
    S2 = jnp.sum(e2 * d, axis=1, keepdims=True)
    s_row = S1 / s1 - S2 / s2

    acc_ref[i, 0:1, :] = loss_r.T
    acc_ref[i, 1:2, :] = dc_r.T
    acc_ref[i, 2:3, :] = cond_r.astype(jnp.float32).T
    acc_ref[i, 3:4, :] = s_row.T

    @pl.when(i == G - 1)
    def _selection():
        RR = 128
        CC = (G * _R) // RR
        loss = acc_ref[:, 0, :].reshape(RR, CC)
        n_f = jnp.float32(RR * CC)

        mean_v = jnp.sum(loss) / n_f
        n_small = jnp.sum((loss < mean_v).astype(jnp.int32))
        k = jnp.maximum(kfloor_ref[0, 0], n_small)

        bits = jax.lax.bitcast_convert_type(loss, jnp.int32)

        def body(_, carry):
            lo, hi = carry
            mid = lo + (hi - lo) // 2
            c = jnp.sum((bits <= mid).astype(jnp.int32))
            pred = c >= k
            return jnp.where(pred, lo, mid + 1), jnp.where(pred, mid, hi)

        _, vbits = jax.lax.fori_loop(
            0, 31, body, (jnp.int32(0), jnp.int32(0x7F800000))
        )

        count_less = jnp.sum((bits < vbits).astype(jnp.int32))
        eq = bits == vbits
        need = (k - count_less).astype(jnp.float32)

        eqf = eq.astype(jnp.float32)
        r0 = jax.lax.broadcasted_iota(jnp.int32, (RR, CC), 0)
        c1 = jax.lax.broadcasted_iota(jnp.int32, (RR, CC), 1)
        lower_incl = (r0 <= c1).astype(jnp.float32)
        strict_lower = (c1 < r0).astype(jnp.float32)
        incl = jnp.dot(eqf, lower_incl, preferred_element_type=jnp.float32)
        rowtot = incl[:, CC - 1 : CC]
        offs = jnp.dot(strict_lower, rowtot, preferred_element_type=jnp.float32)
        eq_before = incl - eqf + offs

        in_update = (bits < vbits) | (eq & (eq_before < need))

        flat_idx = r0 * CC + c1
        upd1 = jnp.logical_and(jnp.logical_not(in_update), flat_idx >= 1)
        condb = acc_ref[:, 2, :].reshape(RR, CC) > 0.5

        loss_clean = jnp.sum(jnp.where(in_update, loss, 0.0))
        loss_dc = jnp.sum(
            jnp.where(
                jnp.logical_and(upd1, condb),
                acc_ref[:, 1, :].reshape(RR, CC), 0.0,
            )
        )
        loss1 = jnp.sum(
            jnp.where(jnp.logical_and(upd1, jnp.logical_not(condb)), loss, 0.0)
        )
        inter = jnp.sum(acc_ref[:, 3, :])

        out_ref[0, 0] = (
            loss_clean + loss_dc + _DECAY_W * loss1
        ) / n_f + _CO_LAMBDA * (inter / n_f)


def kernel(y_1, y_2, t, epoch):
    N, C = y_1.shape
    G = N // _R

    t3 = t.reshape(G, _R, 1)
    remember_rate = 1.0 - _INCREMENT * epoch
    kfloor = jnp.floor(remember_rate * N).astype(jnp.int32).reshape(1, 1)

    out = pl.pallas_call(
        _fused_kernel,
        grid=(G,),
        in_specs=[
            pl.BlockSpec((_R, C), lambda i: (i, 0)),
            pl.BlockSpec((_R, C), lambda i: (i, 0)),
            pl.BlockSpec((1, _R, 1), lambda i: (i, 0, 0)),
            pl.BlockSpec(memory_space=pltpu.SMEM),
        ],
        out_specs=pl.BlockSpec(memory_space=pltpu.SMEM),
        out_shape=jax.ShapeDtypeStruct((1, 1), jnp.float32),
        scratch_shapes=[pltpu.VMEM((G, 4, _R), jnp.float32)],
    )(y_1, y_2, t3, kfloor)

    return out.reshape(())
